# baseline (speedup 1.0000x reference)
import jax
import jax.numpy as jnp
from jax import lax
from jax.experimental import pallas as pl
from jax.experimental.pallas import tpu as pltpu
from jax.experimental.pallas import tpu_sc as plsc

OH_, OW_ = 384, 384
B_, C_ = 2, 96
HW = 384 * 384
NP = B_ * C_
NC, NS, L = 2, 16, 16
PX = HW // NS
NQ = 4
QPX = PX // NQ
QG = QPX // L
NPC = NP // NC


def _sc_body(x_hbm, sm_hbm, out_hbm,
             idx_q0, idx_q1, idx_q2, idx_q3, c_q0, c_q1, c_q2, c_q3,
             x_v0, x_v1, wx_v, wy_v, z_v, table,
             s_x0, s_x1, s_sc, s_dr):
    c = lax.axis_index("c")
    s = lax.axis_index("s")
    px_base = s * PX
    base = c * NPC
    idx_qs = [idx_q0, idx_q1, idx_q2, idx_q3]
    c_qs = [c_q0, c_q1, c_q2, c_q3]

    pltpu.sync_copy(sm_hbm.at[pl.ds(2 * px_base, PX)], x_v0)
    pltpu.sync_copy(sm_hbm.at[pl.ds(2 * px_base + PX, PX)], x_v1)
    lanes = lax.iota(jnp.int32, L)
    perm_e = (lanes * 2) % L
    perm_o = (lanes * 2 + 1) % L
    lo_half = lanes < (L // 2)

    def init_group(q):
        idx_b = idx_qs[q]
        src_v = x_v0 if q < 2 else x_v1
        def body(i, carry):
            g = q * QG + i
            pair_base = ((q % 2) * QG + i) * 2 * L
            v_a = src_v[pl.ds(pair_base, L)]
            v_b = src_v[pl.ds(pair_base + L, L)]
            mx = jnp.where(lo_half,
                           v_a.at[perm_e].get(mode="promise_in_bounds"),
                           v_b.at[perm_e].get(mode="promise_in_bounds"))
            my = jnp.where(lo_half,
                           v_a.at[perm_o].get(mode="promise_in_bounds"),
                           v_b.at[perm_o].get(mode="promise_in_bounds"))
            x0i = mx.astype(jnp.int32)
            y0i = my.astype(jnp.int32)
            wx1 = mx - x0i.astype(jnp.float32)
            wy1 = my - y0i.astype(jnp.float32)
            zero = jnp.zeros((L,), jnp.int32)
            maxw = jnp.full((L,), OW_ - 1, jnp.int32)
            maxh = jnp.full((L,), OH_ - 1, jnp.int32)
            x0 = jnp.minimum(jnp.maximum(x0i, zero), maxw)
            x1 = jnp.minimum(jnp.maximum(x0i + 1, zero), maxw)
            y0 = jnp.minimum(jnp.maximum(y0i, zero), maxh)
            y1 = jnp.minimum(jnp.maximum(y0i + 1, zero), maxh)
            fl = i * (4 * L)
            idx_b[pl.ds(fl, L)] = y0 * OW_ + x0
            idx_b[pl.ds(fl + L, L)] = y0 * OW_ + x1
            idx_b[pl.ds(fl + 2 * L, L)] = y1 * OW_ + x0
            idx_b[pl.ds(fl + 3 * L, L)] = y1 * OW_ + x1
            off = g * L
            wx_v[pl.ds(off, L)] = wx1
            wy_v[pl.ds(off, L)] = wy1
            z_v[pl.ds(off, L)] = jnp.zeros((L,), jnp.float32)
            return carry
        return body

    for q in range(NQ):
        lax.fori_loop(0, QG, init_group(q), 0)

    pltpu.sync_copy(z_v, table.at[pl.ds(px_base, PX)])
    pltpu.async_copy(table.at[pl.ds(px_base, PX)],
                     out_hbm.at[base, pl.ds(px_base, PX)], s_dr)

    pltpu.async_copy(x_hbm.at[base, pl.ds(px_base, PX)], x_v0, s_x0)
    pltpu.async_copy(x_hbm.at[base + 1, pl.ds(px_base, PX)], x_v1, s_x1)

    def do_plane(plane, x_v, s_x):
        pltpu.make_async_copy(
            x_hbm.at[plane, pl.ds(px_base, PX)], x_v, s_x).wait()

        def comp_quarter(q):
            c_b = c_qs[q]
            def body(i, carry):
                off = (q * QG + i) * L
                fl = i * (4 * L)
                v = x_v[pl.ds(off, L)]
                wx1 = wx_v[pl.ds(off, L)]
                wy1 = wy_v[pl.ds(off, L)]
                one = jnp.ones((L,), jnp.float32)
                vy0 = v * (one - wy1)
                vy1 = v * wy1
                c_b[pl.ds(fl, L)] = vy0 * (one - wx1)
                c_b[pl.ds(fl + L, L)] = vy0 * wx1
                c_b[pl.ds(fl + 2 * L, L)] = vy1 * (one - wx1)
                c_b[pl.ds(fl + 3 * L, L)] = vy1 * wx1
                return carry
            lax.fori_loop(0, QG, body, 0)

        comp_quarter(0)
        pltpu.make_async_copy(
            table.at[pl.ds(px_base, PX)],
            out_hbm.at[0, pl.ds(px_base, PX)],
            s_dr).wait()
        pltpu.sync_copy(z_v, table.at[pl.ds(px_base, PX)])
        plsc.subcore_barrier()

        descs = [pltpu.async_copy(c_qs[0], table.at[idx_qs[0]],
                                  s_sc, add=True)]
        for q in range(1, NQ):
            comp_quarter(q)
            descs.append(pltpu.async_copy(c_qs[q], table.at[idx_qs[q]],
                                          s_sc, add=True))
        pltpu.async_copy(
            x_hbm.at[(plane + 2) % NP, pl.ds(px_base, PX)],
            x_v, s_x)
        for d in descs:
            d.wait()
        plsc.subcore_barrier()
        pltpu.async_copy(table.at[pl.ds(px_base, PX)],
                         out_hbm.at[plane, pl.ds(px_base, PX)], s_dr)

    def plane_pair(i, carry):
        do_plane(base + 2 * i, x_v0, s_x0)
        do_plane(base + 2 * i + 1, x_v1, s_x1)
        return carry

    lax.fori_loop(0, NPC // 2, plane_pair, 0)

    pltpu.make_async_copy(
        table.at[pl.ds(px_base, PX)],
        out_hbm.at[base + NPC - 1, pl.ds(px_base, PX)], s_dr).wait()
    pltpu.make_async_copy(
        x_hbm.at[0, pl.ds(px_base, PX)], x_v0, s_x0).wait()
    pltpu.make_async_copy(
        x_hbm.at[0, pl.ds(px_base, PX)], x_v1, s_x1).wait()


@jax.jit
def _splat(x2, sm1):
    mesh = plsc.VectorSubcoreMesh(core_axis_name="c", subcore_axis_name="s")
    return pl.kernel(
        _sc_body,
        out_type=jax.ShapeDtypeStruct((NP, HW), jnp.float32),
        mesh=mesh,
        scratch_types=[
            pltpu.VMEM((4 * QPX,), jnp.int32),
            pltpu.VMEM((4 * QPX,), jnp.int32),
            pltpu.VMEM((4 * QPX,), jnp.int32),
            pltpu.VMEM((4 * QPX,), jnp.int32),
            pltpu.VMEM((4 * QPX,), jnp.float32),
            pltpu.VMEM((4 * QPX,), jnp.float32),
            pltpu.VMEM((4 * QPX,), jnp.float32),
            pltpu.VMEM((4 * QPX,), jnp.float32),
            pltpu.VMEM((PX,), jnp.float32),
            pltpu.VMEM((PX,), jnp.float32),
            pltpu.VMEM((PX,), jnp.float32),
            pltpu.VMEM((PX,), jnp.float32),
            pltpu.VMEM((PX,), jnp.float32),
            pltpu.VMEM_SHARED((HW,), jnp.float32),
            pltpu.SemaphoreType.DMA,
            pltpu.SemaphoreType.DMA,
            pltpu.SemaphoreType.DMA,
            pltpu.SemaphoreType.DMA,
        ],
    )(x2, sm1)


def kernel(x, sample_map, output_height, output_width):
    Bn, Cn, Hn, Wn = x.shape
    x2 = x.reshape(Bn * Cn, Hn * Wn)
    sm1 = sample_map.reshape(-1)
    out2 = _splat(x2, sm1)
    return out2.reshape(Bn, Cn, OH_, OW_)


# speedup vs baseline: 1.0409x; 1.0409x over previous
import jax
import jax.numpy as jnp
from jax import lax
from jax.experimental import pallas as pl
from jax.experimental.pallas import tpu as pltpu
from jax.experimental.pallas import tpu_sc as plsc

OH_, OW_ = 384, 384
B_, C_ = 2, 96
HW = 384 * 384
NP = B_ * C_
NC, NS, L = 2, 16, 16
PX = HW // NS
NQ = 4
QPX = PX // NQ
QG = QPX // L
NPC = NP // NC


def _sc_body(x4_hbm, sm_hbm, out_hbm,
             idx_q0, idx_q1, idx_q2, idx_q3, c_q0, c_q1, c_q2, c_q3,
             x_v0, x_v1, wx_v, wy_v, z_v, table,
             s_x0, s_x1, s_sc, s_dr):
    c = lax.axis_index("c")
    s = lax.axis_index("s")
    px_base = s * PX
    base = c * NPC
    idx_qs = [idx_q0, idx_q1, idx_q2, idx_q3]
    c_qs = [c_q0, c_q1, c_q2, c_q3]

    lanes = lax.iota(jnp.int32, L)
    perm_e = (lanes * 2) % L
    perm_o = (lanes * 2 + 1) % L
    lo_half = lanes < (L // 2)

    def init_group(q):
        idx_b = idx_qs[q]
        src_v = z_v
        def body(i, carry):
            g = q * QG + i
            pair_base = ((q % 2) * QG + i) * 2 * L
            v_a = src_v[pl.ds(pair_base, L)]
            v_b = src_v[pl.ds(pair_base + L, L)]
            mx = jnp.where(lo_half,
                           v_a.at[perm_e].get(mode="promise_in_bounds"),
                           v_b.at[perm_e].get(mode="promise_in_bounds"))
            my = jnp.where(lo_half,
                           v_a.at[perm_o].get(mode="promise_in_bounds"),
                           v_b.at[perm_o].get(mode="promise_in_bounds"))
            x0i = mx.astype(jnp.int32)
            y0i = my.astype(jnp.int32)
            wx1 = mx - x0i.astype(jnp.float32)
            wy1 = my - y0i.astype(jnp.float32)
            zero = jnp.zeros((L,), jnp.int32)
            maxw = jnp.full((L,), OW_ - 1, jnp.int32)
            maxh = jnp.full((L,), OH_ - 1, jnp.int32)
            x0 = jnp.minimum(jnp.maximum(x0i, zero), maxw)
            x1 = jnp.minimum(jnp.maximum(x0i + 1, zero), maxw)
            y0 = jnp.minimum(jnp.maximum(y0i, zero), maxh)
            y1 = jnp.minimum(jnp.maximum(y0i + 1, zero), maxh)
            fl = i * (4 * L)
            idx_b[pl.ds(fl, L)] = y0 * OW_ + x0
            idx_b[pl.ds(fl + L, L)] = y0 * OW_ + x1
            idx_b[pl.ds(fl + 2 * L, L)] = y1 * OW_ + x0
            idx_b[pl.ds(fl + 3 * L, L)] = y1 * OW_ + x1
            off = g * L
            wx_v[pl.ds(off, L)] = wx1
            wy_v[pl.ds(off, L)] = wy1
            return carry
        return body

    for q in range(NQ):
        if q % 2 == 0:
            pltpu.sync_copy(
                sm_hbm.at[pl.ds(2 * px_base + (q // 2) * PX, PX)], z_v)
        lax.fori_loop(0, QG, init_group(q), 0)

    def zero_fill(i, carry):
        z_v[pl.ds(i * L, L)] = jnp.zeros((L,), jnp.float32)
        return carry

    lax.fori_loop(0, PX // L, zero_fill, 0)

    pltpu.sync_copy(z_v, table.at[pl.ds(px_base, PX)])
    for row in range(24):
        pltpu.async_copy(
            table.at[pl.ds(px_base + row * OW_, OW_)],
            out_hbm.at[base // C_, base % C_, s * 24 + row, :], s_dr)

    pltpu.async_copy(
        x4_hbm.at[base // C_, base % C_, pl.ds(s * 24, 24), :], x_v0, s_x0)
    pltpu.async_copy(
        x4_hbm.at[(base + 1) // C_, (base + 1) % C_, pl.ds(s * 24, 24), :],
        x_v1, s_x1)

    def do_plane(plane, x_v, s_x):
        b_i = plane // C_
        ch = plane % C_
        pltpu.make_async_copy(
            x4_hbm.at[b_i, ch, pl.ds(s * 24, 24), :], x_v, s_x).wait()

        def comp_quarter(q):
            c_b = c_qs[q]
            def body(i, carry):
                off = (q * QG + i) * L
                fl = i * (4 * L)
                g = q * QG + i
                v = x_v[g // 24, pl.ds((g % 24) * L, L)]
                wx1 = wx_v[pl.ds(off, L)]
                wy1 = wy_v[pl.ds(off, L)]
                one = jnp.ones((L,), jnp.float32)
                vy0 = v * (one - wy1)
                vy1 = v * wy1
                c_b[pl.ds(fl, L)] = vy0 * (one - wx1)
                c_b[pl.ds(fl + L, L)] = vy0 * wx1
                c_b[pl.ds(fl + 2 * L, L)] = vy1 * (one - wx1)
                c_b[pl.ds(fl + 3 * L, L)] = vy1 * wx1
                return carry
            lax.fori_loop(0, QG, body, 0)

        comp_quarter(0)
        for row in range(24):
            pltpu.make_async_copy(
                table.at[pl.ds(row * OW_, OW_)],
                out_hbm.at[0, 0, row, :],
                s_dr).wait()
        pltpu.sync_copy(z_v, table.at[pl.ds(px_base, PX)])
        plsc.subcore_barrier()

        descs = [pltpu.async_copy(c_qs[0], table.at[idx_qs[0]],
                                  s_sc, add=True)]
        for q in range(1, NQ):
            comp_quarter(q)
            descs.append(pltpu.async_copy(c_qs[q], table.at[idx_qs[q]],
                                          s_sc, add=True))
        pf = (plane + 2) % NP
        pltpu.async_copy(
            x4_hbm.at[pf // C_, pf % C_, pl.ds(s * 24, 24), :], x_v, s_x)
        for d in descs:
            d.wait()
        plsc.subcore_barrier()
        for row in range(24):
            pltpu.async_copy(
                table.at[pl.ds(px_base + row * OW_, OW_)],
                out_hbm.at[b_i, ch, s * 24 + row, :], s_dr)

    def plane_pair(i, carry):
        do_plane(base + 2 * i, x_v0, s_x0)
        do_plane(base + 2 * i + 1, x_v1, s_x1)
        return carry

    lax.fori_loop(0, NPC // 2, plane_pair, 0)

    for row in range(24):
        pltpu.make_async_copy(
            table.at[pl.ds(row * OW_, OW_)],
            out_hbm.at[0, 0, row, :], s_dr).wait()
    pltpu.make_async_copy(
        x4_hbm.at[0, 0, pl.ds(s * 24, 24), :], x_v0, s_x0).wait()
    pltpu.make_async_copy(
        x4_hbm.at[0, 0, pl.ds(s * 24, 24), :], x_v1, s_x1).wait()


@jax.jit
def _splat(x4, sm1):
    mesh = plsc.VectorSubcoreMesh(core_axis_name="c", subcore_axis_name="s")
    return pl.kernel(
        _sc_body,
        out_type=jax.ShapeDtypeStruct((B_, C_, OH_, OW_), jnp.float32),
        mesh=mesh,
        compiler_params=pltpu.CompilerParams(use_tc_tiling_on_sc=True),
        scratch_types=[
            pltpu.VMEM((4 * QPX,), jnp.int32),
            pltpu.VMEM((4 * QPX,), jnp.int32),
            pltpu.VMEM((4 * QPX,), jnp.int32),
            pltpu.VMEM((4 * QPX,), jnp.int32),
            pltpu.VMEM((4 * QPX,), jnp.float32),
            pltpu.VMEM((4 * QPX,), jnp.float32),
            pltpu.VMEM((4 * QPX,), jnp.float32),
            pltpu.VMEM((4 * QPX,), jnp.float32),
            pltpu.VMEM((24, 384), jnp.float32),
            pltpu.VMEM((24, 384), jnp.float32),
            pltpu.VMEM((PX,), jnp.float32),
            pltpu.VMEM((PX,), jnp.float32),
            pltpu.VMEM((PX,), jnp.float32),
            pltpu.VMEM_SHARED((HW,), jnp.float32),
            pltpu.SemaphoreType.DMA,
            pltpu.SemaphoreType.DMA,
            pltpu.SemaphoreType.DMA,
            pltpu.SemaphoreType.DMA,
        ],
    )(x4, sm1)


def kernel(x, sample_map, output_height, output_width):
    sm1 = sample_map.reshape(-1)
    return _splat(x, sm1)


# speedup vs baseline: 1.0935x; 1.0506x over previous
import jax
import jax.numpy as jnp
from jax import lax
from jax.experimental import pallas as pl
from jax.experimental.pallas import tpu as pltpu
from jax.experimental.pallas import tpu_sc as plsc

OH_, OW_ = 384, 384
B_, C_ = 2, 96
HW = 384 * 384
NP = B_ * C_
NC, NS, L = 2, 16, 16
PX = HW // NS
NQ = 4
QPX = PX // NQ
QG = QPX // L
NPC = NP // NC


def _sc_body(x_hbm, sm_hbm, out_hbm,
             idx_q0, idx_q1, idx_q2, idx_q3, c_q0, c_q1, c_q2, c_q3,
             x_v, wx_v, wy_v, z_v, t0, t1,
             s_x, s_sc, s_dr):
    c = lax.axis_index("c")
    s = lax.axis_index("s")
    px_base = s * PX
    base = c * NPC
    idx_qs = [idx_q0, idx_q1, idx_q2, idx_q3]
    c_qs = [c_q0, c_q1, c_q2, c_q3]

    lanes = lax.iota(jnp.int32, L)
    perm_e = (lanes * 2) % L
    perm_o = (lanes * 2 + 1) % L
    lo_half = lanes < (L // 2)

    def init_group(q):
        idx_b = idx_qs[q]

        def body(i, carry):
            g = q * QG + i
            pair_base = ((q % 2) * QG + i) * 2 * L
            v_a = c_q0[pl.ds(pair_base, L)]
            v_b = c_q0[pl.ds(pair_base + L, L)]
            mx = jnp.where(lo_half,
                           v_a.at[perm_e].get(mode="promise_in_bounds"),
                           v_b.at[perm_e].get(mode="promise_in_bounds"))
            my = jnp.where(lo_half,
                           v_a.at[perm_o].get(mode="promise_in_bounds"),
                           v_b.at[perm_o].get(mode="promise_in_bounds"))
            x0i = mx.astype(jnp.int32)
            y0i = my.astype(jnp.int32)
            wx1 = mx - x0i.astype(jnp.float32)
            wy1 = my - y0i.astype(jnp.float32)
            zero = jnp.zeros((L,), jnp.int32)
            maxw = jnp.full((L,), OW_ - 1, jnp.int32)
            maxh = jnp.full((L,), OH_ - 1, jnp.int32)
            x0 = jnp.minimum(jnp.maximum(x0i, zero), maxw)
            x1 = jnp.minimum(jnp.maximum(x0i + 1, zero), maxw)
            y0 = jnp.minimum(jnp.maximum(y0i, zero), maxh)
            y1 = jnp.minimum(jnp.maximum(y0i + 1, zero), maxh)
            fl = i * (4 * L)
            idx_b[pl.ds(fl, L)] = y0 * OW_ + x0
            idx_b[pl.ds(fl + L, L)] = y0 * OW_ + x1
            idx_b[pl.ds(fl + 2 * L, L)] = y1 * OW_ + x0
            idx_b[pl.ds(fl + 3 * L, L)] = y1 * OW_ + x1
            off = g * L
            wx_v[pl.ds(off, L)] = wx1
            wy_v[pl.ds(off, L)] = wy1
            return carry

        return body

    for q in range(NQ):
        if q % 2 == 0:
            pltpu.sync_copy(
                sm_hbm.at[pl.ds(2 * px_base + (q // 2) * PX, PX)],
                c_q0.at[pl.ds(0, PX)])
        lax.fori_loop(0, QG, init_group(q), 0)

    def zero_fill(i, carry):
        z_v[pl.ds(i * L, L)] = jnp.zeros((L,), jnp.float32)
        return carry

    lax.fori_loop(0, PX // L, zero_fill, 0)

    for q in range(NQ):
        pltpu.async_copy(c_qs[q], t1.at[idx_qs[q]], s_sc, add=True)
    pltpu.async_copy(z_v, t1.at[pl.ds(px_base, PX)], s_dr)
    pltpu.async_copy(z_v, t1.at[pl.ds(px_base, PX)], s_dr)
    pltpu.async_copy(x_hbm.at[base, pl.ds(px_base, PX)], x_v, s_x)

    def drain_plane(plane, tbl):
        pltpu.async_copy(tbl.at[pl.ds(px_base, PX)],
                         out_hbm.at[plane, pl.ds(px_base, PX)], s_dr)

    def do_plane(pp, tbl, tbl_prev, drain_prev):
        plane = base + pp
        pltpu.make_async_copy(
            x_hbm.at[0, pl.ds(px_base, PX)], x_v, s_x).wait()

        def comp_quarter(q):
            c_b = c_qs[q]

            def body(i, carry):
                g = q * QG + i
                off = g * L
                fl = i * (4 * L)
                v = x_v[pl.ds(off, L)]
                wx1 = wx_v[pl.ds(off, L)]
                wy1 = wy_v[pl.ds(off, L)]
                one = jnp.ones((L,), jnp.float32)
                vy0 = v * (one - wy1)
                vy1 = v * wy1
                c_b[pl.ds(fl, L)] = vy0 * (one - wx1)
                c_b[pl.ds(fl + L, L)] = vy0 * wx1
                c_b[pl.ds(fl + 2 * L, L)] = vy1 * (one - wx1)
                c_b[pl.ds(fl + 3 * L, L)] = vy1 * wx1
                return carry

            lax.fori_loop(0, QG, body, 0)

        for q in range(NQ):
            pltpu.make_async_copy(c_qs[q], t1.at[idx_qs[q]], s_sc).wait()
            comp_quarter(q)
            if q == 0:
                pltpu.make_async_copy(
                    t1.at[pl.ds(px_base, PX)],
                    out_hbm.at[0, pl.ds(px_base, PX)], s_dr).wait()
                pltpu.sync_copy(z_v, tbl.at[pl.ds(px_base, PX)])
                plsc.subcore_barrier()
            pltpu.async_copy(c_qs[q], tbl.at[idx_qs[q]], s_sc, add=True)

        pf = (plane + 1) % NP
        pltpu.async_copy(x_hbm.at[pf, pl.ds(px_base, PX)], x_v, s_x)

        plsc.subcore_barrier()

        if drain_prev:
            drain_plane(plane - 1, tbl_prev)

    do_plane(0, t0, t1, False)

    def plane_pair(j, carry):
        do_plane(2 * j + 1, t1, t0, True)
        do_plane(2 * j + 2, t0, t1, True)
        return carry

    lax.fori_loop(0, (NPC - 2) // 2, plane_pair, 0)
    do_plane(NPC - 1, t1, t0, True)

    for q in range(NQ):
        pltpu.make_async_copy(c_qs[q], t1.at[idx_qs[q]], s_sc).wait()
    plsc.subcore_barrier()
    drain_plane(base + NPC - 1, t1)
    pltpu.make_async_copy(t1.at[pl.ds(px_base, PX)],
                          out_hbm.at[0, pl.ds(px_base, PX)], s_dr).wait()
    pltpu.make_async_copy(t1.at[pl.ds(px_base, PX)],
                          out_hbm.at[0, pl.ds(px_base, PX)], s_dr).wait()
    pltpu.make_async_copy(
        x_hbm.at[0, pl.ds(px_base, PX)], x_v, s_x).wait()


@jax.jit
def _splat(x2, sm1):
    mesh = plsc.VectorSubcoreMesh(core_axis_name="c", subcore_axis_name="s")
    return pl.kernel(
        _sc_body,
        out_type=jax.ShapeDtypeStruct((NP, HW), jnp.float32),
        mesh=mesh,
        scratch_types=[
            pltpu.VMEM((4 * QPX,), jnp.int32),
            pltpu.VMEM((4 * QPX,), jnp.int32),
            pltpu.VMEM((4 * QPX,), jnp.int32),
            pltpu.VMEM((4 * QPX,), jnp.int32),
            pltpu.VMEM((4 * QPX,), jnp.float32),
            pltpu.VMEM((4 * QPX,), jnp.float32),
            pltpu.VMEM((4 * QPX,), jnp.float32),
            pltpu.VMEM((4 * QPX,), jnp.float32),
            pltpu.VMEM((PX,), jnp.float32),
            pltpu.VMEM((PX,), jnp.float32),
            pltpu.VMEM((PX,), jnp.float32),
            pltpu.VMEM((PX,), jnp.float32),
            pltpu.VMEM_SHARED((HW,), jnp.float32),
            pltpu.VMEM_SHARED((HW,), jnp.float32),
            pltpu.SemaphoreType.DMA,
            pltpu.SemaphoreType.DMA,
            pltpu.SemaphoreType.DMA,
        ],
    )(x2, sm1)


def kernel(x, sample_map, output_height, output_width):
    Bn, Cn, Hn, Wn = x.shape
    x2 = x.reshape(Bn * Cn, Hn * Wn)
    sm1 = sample_map.reshape(-1)
    out2 = _splat(x2, sm1)
    return out2.reshape(Bn, Cn, OH_, OW_)


# speedup vs baseline: 1.2172x; 1.1131x over previous
import jax
import jax.numpy as jnp
from jax import lax
from jax.experimental import pallas as pl
from jax.experimental.pallas import tpu as pltpu
from jax.experimental.pallas import tpu_sc as plsc

OH_, OW_ = 384, 384
B_, C_ = 2, 96
HW = 384 * 384
NP = B_ * C_
NC, NS, L = 2, 16, 16
PX = HW // NS
NQ = 4
QPX = PX // NQ
QG = QPX // L
NPC = NP // NC
ROWS = OH_ // NS


def _sc_body(x4_hbm, sm_hbm, out_hbm,
             idx_q0, idx_q1, idx_q2, idx_q3, c_q0, c_q1, c_q2, c_q3,
             x_v, wx_v, wy_v, z_v, t0, t1,
             s_x, s_sc, s_dr):
    c = lax.axis_index("c")
    s = lax.axis_index("s")
    px_base = s * PX
    base = c * NPC
    idx_qs = [idx_q0, idx_q1, idx_q2, idx_q3]
    c_qs = [c_q0, c_q1, c_q2, c_q3]

    lanes = lax.iota(jnp.int32, L)
    perm_e = (lanes * 2) % L
    perm_o = (lanes * 2 + 1) % L
    lo_half = lanes < (L // 2)

    def init_group(q):
        idx_b = idx_qs[q]

        def body(i, carry):
            g = q * QG + i
            pair_base = ((q % 2) * QG + i) * 2 * L
            v_a = z_v[pl.ds(pair_base, L)]
            v_b = z_v[pl.ds(pair_base + L, L)]
            mx = jnp.where(lo_half,
                           v_a.at[perm_e].get(mode="promise_in_bounds"),
                           v_b.at[perm_e].get(mode="promise_in_bounds"))
            my = jnp.where(lo_half,
                           v_a.at[perm_o].get(mode="promise_in_bounds"),
                           v_b.at[perm_o].get(mode="promise_in_bounds"))
            x0i = mx.astype(jnp.int32)
            y0i = my.astype(jnp.int32)
            wx1 = mx - x0i.astype(jnp.float32)
            wy1 = my - y0i.astype(jnp.float32)
            zero = jnp.zeros((L,), jnp.int32)
            maxw = jnp.full((L,), OW_ - 1, jnp.int32)
            maxh = jnp.full((L,), OH_ - 1, jnp.int32)
            x0 = jnp.minimum(jnp.maximum(x0i, zero), maxw)
            x1 = jnp.minimum(jnp.maximum(x0i + 1, zero), maxw)
            y0 = jnp.minimum(jnp.maximum(y0i, zero), maxh)
            y1 = jnp.minimum(jnp.maximum(y0i + 1, zero), maxh)
            fl = i * (4 * L)
            idx_b[pl.ds(fl, L)] = y0 * OW_ + x0
            idx_b[pl.ds(fl + L, L)] = y0 * OW_ + x1
            idx_b[pl.ds(fl + 2 * L, L)] = y1 * OW_ + x0
            idx_b[pl.ds(fl + 3 * L, L)] = y1 * OW_ + x1
            off = g * L
            wx_v[pl.ds(off, L)] = wx1
            wy_v[pl.ds(off, L)] = wy1
            return carry

        return body

    for q in range(NQ):
        if q % 2 == 0:
            pltpu.sync_copy(
                sm_hbm.at[pl.ds(2 * px_base + (q // 2) * PX, PX)], z_v)
        lax.fori_loop(0, QG, init_group(q), 0)

    def zero_fill(i, carry):
        z_v[pl.ds(i * L, L)] = jnp.zeros((L,), jnp.float32)
        return carry

    lax.fori_loop(0, PX // L, zero_fill, 0)

    for q in range(NQ):
        pltpu.async_copy(c_qs[q], t1.at[idx_qs[q]], s_sc, add=True)
    pltpu.async_copy(z_v, t1.at[pl.ds(px_base, PX)], s_dr)
    pltpu.async_copy(z_v, t1.at[pl.ds(px_base, PX)], s_dr)
    pltpu.async_copy(
        x4_hbm.at[base // C_, base % C_, pl.ds(s * ROWS, ROWS), :], x_v, s_x)

    def drain_plane(plane, tbl):
        b_i = plane // C_
        ch = plane % C_
        for row in range(ROWS):
            pltpu.async_copy(
                tbl.at[pl.ds(px_base + row * OW_, OW_)],
                out_hbm.at[b_i, ch, s * ROWS + row, :], s_dr)

    def do_plane(pp, tbl, tbl_prev, drain_prev):
        plane = base + pp
        pltpu.make_async_copy(
            x4_hbm.at[0, 0, pl.ds(s * ROWS, ROWS), :], x_v, s_x).wait()

        def comp_quarter(q):
            c_b = c_qs[q]

            def body(i, carry):
                g = q * QG + i
                off = g * L
                fl = i * (4 * L)
                v = x_v[g // ROWS, pl.ds((g % ROWS) * L, L)]
                wx1 = wx_v[pl.ds(off, L)]
                wy1 = wy_v[pl.ds(off, L)]
                one = jnp.ones((L,), jnp.float32)
                vy0 = v * (one - wy1)
                vy1 = v * wy1
                c_b[pl.ds(fl, L)] = vy0 * (one - wx1)
                c_b[pl.ds(fl + L, L)] = vy0 * wx1
                c_b[pl.ds(fl + 2 * L, L)] = vy1 * (one - wx1)
                c_b[pl.ds(fl + 3 * L, L)] = vy1 * wx1
                return carry

            lax.fori_loop(0, QG, body, 0)

        for q in range(NQ):
            pltpu.make_async_copy(c_qs[q], t1.at[idx_qs[q]], s_sc).wait()
            comp_quarter(q)
            if q == 0:
                for row in range(ROWS):
                    pltpu.make_async_copy(
                        t1.at[pl.ds(row * OW_, OW_)],
                        out_hbm.at[0, 0, row, :], s_dr).wait()
                pltpu.sync_copy(z_v, tbl.at[pl.ds(px_base, PX)])
                plsc.subcore_barrier()
            pltpu.async_copy(c_qs[q], tbl.at[idx_qs[q]], s_sc, add=True)

        pf = (plane + 1) % NP
        pltpu.async_copy(
            x4_hbm.at[pf // C_, pf % C_, pl.ds(s * ROWS, ROWS), :], x_v, s_x)

        plsc.subcore_barrier()

        if drain_prev:
            drain_plane(plane - 1, tbl_prev)

    do_plane(0, t0, t1, False)

    def plane_pair(j, carry):
        do_plane(2 * j + 1, t1, t0, True)
        do_plane(2 * j + 2, t0, t1, True)
        return carry

    lax.fori_loop(0, (NPC - 2) // 2, plane_pair, 0)
    do_plane(NPC - 1, t1, t0, True)

    for q in range(NQ):
        pltpu.make_async_copy(c_qs[q], t1.at[idx_qs[q]], s_sc).wait()
    plsc.subcore_barrier()
    drain_plane(base + NPC - 1, t1)
    for row in range(2 * ROWS):
        pltpu.make_async_copy(
            t1.at[pl.ds((row % ROWS) * OW_, OW_)],
            out_hbm.at[0, 0, row % ROWS, :], s_dr).wait()
    pltpu.make_async_copy(
        x4_hbm.at[0, 0, pl.ds(s * ROWS, ROWS), :], x_v, s_x).wait()


@jax.jit
def _splat(x4, sm1):
    mesh = plsc.VectorSubcoreMesh(core_axis_name="c", subcore_axis_name="s")
    return pl.kernel(
        _sc_body,
        out_type=jax.ShapeDtypeStruct((B_, C_, OH_, OW_), jnp.float32),
        mesh=mesh,
        compiler_params=pltpu.CompilerParams(use_tc_tiling_on_sc=True),
        scratch_types=[
            pltpu.VMEM((4 * QPX,), jnp.int32),
            pltpu.VMEM((4 * QPX,), jnp.int32),
            pltpu.VMEM((4 * QPX,), jnp.int32),
            pltpu.VMEM((4 * QPX,), jnp.int32),
            pltpu.VMEM((4 * QPX,), jnp.float32),
            pltpu.VMEM((4 * QPX,), jnp.float32),
            pltpu.VMEM((4 * QPX,), jnp.float32),
            pltpu.VMEM((4 * QPX,), jnp.float32),
            pltpu.VMEM((ROWS, OW_), jnp.float32),
            pltpu.VMEM((PX,), jnp.float32),
            pltpu.VMEM((PX,), jnp.float32),
            pltpu.VMEM((PX,), jnp.float32),
            pltpu.VMEM_SHARED((HW,), jnp.float32),
            pltpu.VMEM_SHARED((HW,), jnp.float32),
            pltpu.SemaphoreType.DMA,
            pltpu.SemaphoreType.DMA,
            pltpu.SemaphoreType.DMA,
        ],
    )(x4, sm1)


def kernel(x, sample_map, output_height, output_width):
    sm1 = sample_map.reshape(-1)
    return _splat(x, sm1)
